# Initial kernel scaffold; baseline (speedup 1.0000x reference)
import jax
import jax.numpy as jnp
from jax.experimental import pallas as pl


def kernel(x, edge_index, W1, b1, W2, b2):
    raise NotImplementedError("write your pallas kernel here")


# speedup vs baseline: 15.2408x; 15.2408x over previous
import functools

import jax
import jax.numpy as jnp
from jax import lax
from jax.experimental import pallas as pl
from jax.experimental.pallas import tpu as pltpu
from jax.experimental.pallas import tpu_sc as plsc

_N, _D, _H, _C, _E = 10000, 128, 16, 40, 320000
_NC, _NS = 2, 16
_NW = _NC * _NS
_EB = 128
_CH = 80
_EPAD = _NW * _CH * _EB
_ACC = 10240
_ZR = _ACC // _NS
_OR = _N // _NS


def _seg_body(table, srcr, dstr, out,
              src_idx, dst_idx, rows0, rows1, stage, acc, sem0, sem1):
    cid = lax.axis_index("c")
    sid = lax.axis_index("s")
    wid = cid * _NS + sid

    def _zero(i, c):
        stage[i] = jnp.zeros((_H,), jnp.float32)
        return c
    lax.fori_loop(0, _ZR, _zero, 0)
    pltpu.sync_copy(stage, acc.at[pl.ds(sid * _ZR, _ZR)])
    plsc.subcore_barrier()

    base = wid * _CH
    pltpu.sync_copy(srcr.at[pl.ds(base, _CH)], src_idx)
    pltpu.sync_copy(dstr.at[pl.ds(base, _CH)], dst_idx)

    pltpu.async_copy(table.at[src_idx.at[0]], rows0, sem0)

    def _step(i, c):
        j = 2 * i
        pltpu.async_copy(table.at[src_idx.at[j + 1]], rows1, sem1)
        pltpu.make_async_copy(table.at[src_idx.at[j]], rows0, sem0).wait()
        pltpu.sync_copy(rows0, acc.at[dst_idx.at[j]], add=True)

        @pl.when(j + 2 < _CH)
        def _():
            pltpu.async_copy(table.at[src_idx.at[j + 2]], rows0, sem0)

        pltpu.make_async_copy(table.at[src_idx.at[j + 1]], rows1, sem1).wait()
        pltpu.sync_copy(rows1, acc.at[dst_idx.at[j + 1]], add=True)
        return c

    lax.fori_loop(0, _CH // 2, _step, 0)
    plsc.subcore_barrier()

    pltpu.sync_copy(acc.at[pl.ds(sid * _ZR, _ZR)],
                    out.at[cid, pl.ds(sid * _ZR, _ZR)])


_segsum = functools.partial(
    pl.kernel,
    out_type=jax.ShapeDtypeStruct((_NC, _ACC, _H), jnp.float32),
    mesh=plsc.VectorSubcoreMesh(core_axis_name="c", subcore_axis_name="s"),
    scratch_types=[
        pltpu.VMEM((_CH, _EB), jnp.int32),
        pltpu.VMEM((_CH, _EB), jnp.int32),
        pltpu.VMEM((_EB, _H), jnp.float32),
        pltpu.VMEM((_EB, _H), jnp.float32),
        pltpu.VMEM((_ZR, _H), jnp.float32),
        pltpu.VMEM_SHARED((_ACC, _H), jnp.float32),
        pltpu.SemaphoreType.DMA,
        pltpu.SemaphoreType.DMA,
    ],
    compiler_params=pltpu.CompilerParams(use_tc_tiling_on_sc=False),
)(_seg_body)


def _lin1_body(x_ref, w_ref, o_ref):
    o_ref[...] = jnp.dot(x_ref[...], w_ref[...],
                         preferred_element_type=jnp.float32,
                         precision=lax.Precision.HIGHEST)


def _relu_add_body(y_ref, p_ref, b_ref, o_ref):
    s = y_ref[...] + p_ref[0, :_N] + p_ref[1, :_N] + b_ref[...]
    o_ref[...] = jnp.maximum(s, 0.0)


def _out_body(h_ref, q_ref, w_ref, b_ref, o_ref):
    t = h_ref[...] + q_ref[0, :_N] + q_ref[1, :_N]
    z = jnp.dot(t, w_ref[...], preferred_element_type=jnp.float32,
                precision=lax.Precision.HIGHEST) + b_ref[...]
    z = jnp.maximum(z, 0.0)
    m = jnp.max(z, axis=1, keepdims=True)
    z = z - m
    o_ref[...] = z - jnp.log(jnp.sum(jnp.exp(z), axis=1, keepdims=True))


def kernel(x, edge_index, W1, b1, W2, b2):
    src = edge_index[0]
    dst = edge_index[1]
    pad = _EPAD - _E
    srcr = jnp.concatenate(
        [src, jnp.zeros((pad,), jnp.int32)]).reshape(_NW * _CH, _EB)
    dstr = jnp.concatenate(
        [dst, jnp.full((pad,), _N, jnp.int32)]).reshape(_NW * _CH, _EB)

    y1 = pl.pallas_call(
        _lin1_body,
        out_shape=jax.ShapeDtypeStruct((_N, _H), jnp.float32),
    )(x, W1)

    p = _segsum(y1, srcr, dstr)

    h1 = pl.pallas_call(
        _relu_add_body,
        out_shape=jax.ShapeDtypeStruct((_N, _H), jnp.float32),
    )(y1, p, b1.reshape(1, _H))

    q = _segsum(h1, srcr, dstr)

    out = pl.pallas_call(
        _out_body,
        out_shape=jax.ShapeDtypeStruct((_N, _C), jnp.float32),
    )(h1, q, W2, b2.reshape(1, _C))
    return out
